# Initial kernel scaffold; baseline (speedup 1.0000x reference)
import jax
import jax.numpy as jnp
from jax.experimental import pallas as pl


def kernel(x_real, x_imag, edge_index, eps, W1r, b1r, W2r, b2r, W1i, b1i, W2i, b2i):
    raise NotImplementedError("write your pallas kernel here")


# speedup vs baseline: 2.4947x; 2.4947x over previous
import functools

import jax
import jax.numpy as jnp
from jax import lax
from jax.experimental import pallas as pl
from jax.experimental.pallas import tpu as pltpu
from jax.experimental.pallas import tpu_sc as plsc

_NC = 2
_NS = 16
_C = 128
_G = 16


def _make_agg(n, e, d, n_chunks):
    n_groups = n_chunks // _G
    rows_per_tile = 8 * (n // (8 * _NS))
    tail_rows = n - rows_per_tile * _NS
    mesh = plsc.VectorSubcoreMesh(core_axis_name="c", subcore_axis_name="s")

    @functools.partial(
        pl.kernel,
        out_type=jax.ShapeDtypeStruct((2 * n, d), jnp.float32),
        mesh=mesh,
        scratch_types=[
            pltpu.VMEM_SHARED((n + 8, d), jnp.float32),
            pltpu.VMEM((_G, _C), jnp.int32),
            pltpu.VMEM((_G, _C), jnp.int32),
            pltpu.VMEM((_C, d), jnp.float32),
            pltpu.SemaphoreType.DMA,
        ],
    )
    def agg_kernel(x_hbm, row_hbm, col_hbm, out_hbm, acc, ridx, cidx, gbuf, sem):
        c = lax.axis_index("c")
        s = lax.axis_index("s")
        w = c * _NS + s
        r0 = s * rows_per_tile
        pltpu.sync_copy(
            x_hbm.at[pl.ds(c * n + r0, rows_per_tile)],
            acc.at[pl.ds(r0, rows_per_tile)],
        )
        if tail_rows:
            @pl.when(s == _NS - 1)
            def _seed_tail():
                t0 = rows_per_tile * _NS
                pltpu.sync_copy(
                    x_hbm.at[pl.ds(c * n + t0, tail_rows)],
                    acc.at[pl.ds(t0, tail_rows)],
                )
        plsc.subcore_barrier()

        def group(g, carry):
            pltpu.sync_copy(row_hbm.at[w].at[pl.ds(g * _G, _G)], ridx)
            pltpu.sync_copy(col_hbm.at[w].at[pl.ds(g * _G, _G)], cidx)

            def chunk(j, carry2):
                pltpu.async_copy(x_hbm.at[ridx.at[j]], gbuf, sem).wait()
                pltpu.sync_copy(gbuf, acc.at[cidx.at[j]], add=True)
                return carry2

            return lax.fori_loop(0, _G, chunk, carry)

        lax.fori_loop(0, n_groups, group, 0)
        plsc.subcore_barrier()
        pltpu.sync_copy(
            acc.at[pl.ds(r0, rows_per_tile)],
            out_hbm.at[pl.ds(c * n + r0, rows_per_tile)],
        )
        if tail_rows:
            @pl.when(s == _NS - 1)
            def _out_tail():
                t0 = rows_per_tile * _NS
                pltpu.sync_copy(
                    acc.at[pl.ds(t0, tail_rows)],
                    out_hbm.at[pl.ds(c * n + t0, tail_rows)],
                )

    return agg_kernel


def _mlp_body(hpre_ref, x_ref, eps_ref, w1_ref, b1_ref, w2_ref, b2_ref, out_ref):
    eps = eps_ref[0, 0]
    h = hpre_ref[...] + eps * x_ref[...]
    a = jnp.dot(h, w1_ref[0], preferred_element_type=jnp.float32) + b1_ref[0, 0]
    a = jnp.maximum(a, 0.0)
    out_ref[...] = jnp.dot(a, w2_ref[0], preferred_element_type=jnp.float32) + b2_ref[0, 0]


def kernel(x_real, x_imag, edge_index, eps, W1r, b1r, W2r, b2r, W1i, b1i, W2i, b2i):
    n, d = x_real.shape
    e = edge_index.shape[1]
    n_tiles = _NC * _NS
    per_tile = (2 * e) // n_tiles
    chunk_edges = _C * _G
    per_tile_pad = ((per_tile + chunk_edges - 1) // chunk_edges) * chunk_edges
    n_chunks = per_tile_pad // _C

    row = edge_index[0].astype(jnp.int32)
    col = edge_index[1].astype(jnp.int32)
    x_cat = jnp.concatenate(
        [x_real, x_imag, jnp.zeros((8, d), jnp.float32)], axis=0)
    pad = per_tile_pad - per_tile
    row2 = jnp.concatenate([row, row + n]).reshape(n_tiles, per_tile)
    col2 = jnp.concatenate([col, col]).reshape(n_tiles, per_tile)
    row3 = jnp.pad(row2, ((0, 0), (0, pad)), constant_values=2 * n)
    col3 = jnp.pad(col2, ((0, 0), (0, pad)), constant_values=n)
    row3 = row3.reshape(n_tiles, n_chunks, _C)
    col3 = col3.reshape(n_tiles, n_chunks, _C)

    hpre = _make_agg(n, e, d, n_chunks)(x_cat, row3, col3)

    blk = 2000
    nblk = (2 * n) // blk
    half = nblk // 2
    w1s = jnp.stack([W1r.T, W1i.T])
    b1s = jnp.stack([b1r, b1i]).reshape(2, 1, d)
    w2s = jnp.stack([W2r.T, W2i.T])
    b2s = jnp.stack([b2r, b2i]).reshape(2, 1, d)
    eps_b = jnp.broadcast_to(jnp.reshape(eps, (1, 1)), (8, d))

    out_cat = pl.pallas_call(
        _mlp_body,
        grid=(2, half),
        in_specs=[
            pl.BlockSpec((blk, d), lambda r, i: (r * half + i, 0)),
            pl.BlockSpec((blk, d), lambda r, i: (r * half + i, 0)),
            pl.BlockSpec((8, d), lambda r, i: (0, 0)),
            pl.BlockSpec((1, d, d), lambda r, i: (r, 0, 0)),
            pl.BlockSpec((1, 1, d), lambda r, i: (r, 0, 0)),
            pl.BlockSpec((1, d, d), lambda r, i: (r, 0, 0)),
            pl.BlockSpec((1, 1, d), lambda r, i: (r, 0, 0)),
        ],
        out_specs=pl.BlockSpec((blk, d), lambda r, i: (r * half + i, 0)),
        out_shape=jax.ShapeDtypeStruct((2 * n, d), jnp.float32),
    )(hpre, x_cat[: 2 * n], eps_b, w1s, b1s, w2s, b2s)

    return out_cat[:n], out_cat[n:]


# speedup vs baseline: 5.5911x; 2.2411x over previous
import functools

import jax
import jax.numpy as jnp
from jax import lax
from jax.experimental import pallas as pl
from jax.experimental.pallas import tpu as pltpu
from jax.experimental.pallas import tpu_sc as plsc

_NC = 2
_NS = 16
_C = 128
_G = 16


def _make_agg(n, d, n_chunks):
    h = d // 2
    n_groups = n_chunks // _G
    rows_per_tile = 8 * (n // (8 * _NS))
    tail_rows = n - rows_per_tile * _NS
    mesh = plsc.VectorSubcoreMesh(core_axis_name="c", subcore_axis_name="s")

    @functools.partial(
        pl.kernel,
        out_type=jax.ShapeDtypeStruct((4, n, h), jnp.float32),
        mesh=mesh,
        scratch_types=[
            pltpu.VMEM_SHARED((n, h), jnp.float32),
            pltpu.VMEM_SHARED((n + 8, h), jnp.float32),
            pltpu.VMEM((_G, _C), jnp.int32),
            pltpu.VMEM((_G, _C), jnp.int32),
            pltpu.VMEM((2, _C, h), jnp.float32),
            pltpu.SemaphoreType.DMA,
            pltpu.SemaphoreType.DMA,
        ],
    )
    def agg_kernel(xh_hbm, row_hbm, col_hbm, out_hbm, table, acc, ridx, cidx,
                   gbuf, gsem, ssem):
        c = lax.axis_index("c")
        s = lax.axis_index("s")
        r0 = s * rows_per_tile

        def my_slices(src, dst):
            pltpu.sync_copy(src.at[pl.ds(r0, rows_per_tile)],
                            dst.at[pl.ds(r0, rows_per_tile)])
            if tail_rows:
                @pl.when(s == _NS - 1)
                def _tail():
                    t0 = rows_per_tile * _NS
                    pltpu.sync_copy(src.at[pl.ds(t0, tail_rows)],
                                    dst.at[pl.ds(t0, tail_rows)])

        for p in range(2):
            t = 2 * c + p
            my_slices(xh_hbm.at[t], table)
            my_slices(xh_hbm.at[t], acc)
            plsc.subcore_barrier()

            def group(g, carry):
                pltpu.sync_copy(row_hbm.at[s].at[pl.ds(g * _G, _G)], ridx)
                pltpu.sync_copy(col_hbm.at[s].at[pl.ds(g * _G, _G)], cidx)
                for j in range(_G):
                    slot = j % 2
                    if j >= 2:
                        pltpu.make_async_copy(
                            gbuf.at[slot], acc.at[cidx.at[j - 2]], ssem).wait()
                    pltpu.async_copy(
                        table.at[ridx.at[j]], gbuf.at[slot], gsem)
                    if j >= 1:
                        pltpu.make_async_copy(
                            table.at[ridx.at[j - 1]], gbuf.at[(j - 1) % 2],
                            gsem).wait()
                        pltpu.async_copy(
                            gbuf.at[(j - 1) % 2], acc.at[cidx.at[j - 1]],
                            ssem, add=True)
                last = _G - 1
                pltpu.make_async_copy(
                    table.at[ridx.at[last]], gbuf.at[last % 2], gsem).wait()
                pltpu.async_copy(
                    gbuf.at[last % 2], acc.at[cidx.at[last]], ssem, add=True)
                pltpu.make_async_copy(
                    gbuf.at[(last - 1) % 2], acc.at[cidx.at[last - 1]],
                    ssem).wait()
                pltpu.make_async_copy(
                    gbuf.at[last % 2], acc.at[cidx.at[last]], ssem).wait()
                return carry

            lax.fori_loop(0, n_groups, group, 0)
            plsc.subcore_barrier()
            my_slices(acc, out_hbm.at[t])
            plsc.subcore_barrier()

    return agg_kernel


def _mlp_body(ha_ref, hb_ref, x_ref, eps_ref, w1_ref, b1_ref, w2_ref, b2_ref,
              out_ref):
    eps = eps_ref[0, 0]
    hpre = jnp.concatenate([ha_ref[0], hb_ref[0]], axis=1)
    h = hpre + eps * x_ref[...]
    a = jnp.dot(h, w1_ref[0], preferred_element_type=jnp.float32) + b1_ref[0, 0]
    a = jnp.maximum(a, 0.0)
    out_ref[...] = jnp.dot(a, w2_ref[0], preferred_element_type=jnp.float32) + b2_ref[0, 0]


def kernel(x_real, x_imag, edge_index, eps, W1r, b1r, W2r, b2r, W1i, b1i, W2i, b2i):
    n, d = x_real.shape
    e = edge_index.shape[1]
    hd = d // 2
    per_tile = e // _NS
    chunk_edges = _C * _G
    per_tile_pad = ((per_tile + chunk_edges - 1) // chunk_edges) * chunk_edges
    n_chunks = per_tile_pad // _C
    pad = per_tile_pad - per_tile

    row = edge_index[0].astype(jnp.int32)
    col = edge_index[1].astype(jnp.int32)
    xh = jnp.stack([x_real[:, :hd], x_real[:, hd:], x_imag[:, :hd],
                    x_imag[:, hd:]])
    row3 = jnp.pad(row.reshape(_NS, per_tile), ((0, 0), (0, pad)))
    col3 = jnp.pad(col.reshape(_NS, per_tile), ((0, 0), (0, pad)),
                   constant_values=n)
    row3 = row3.reshape(_NS, n_chunks, _C)
    col3 = col3.reshape(_NS, n_chunks, _C)

    hpre_h = _make_agg(n, d, n_chunks)(xh, row3, col3)
    x_cat = jnp.concatenate([x_real, x_imag], axis=0)

    blk = 2000
    half = n // blk
    w1s = jnp.stack([W1r.T, W1i.T])
    b1s = jnp.stack([b1r, b1i]).reshape(2, 1, d)
    w2s = jnp.stack([W2r.T, W2i.T])
    b2s = jnp.stack([b2r, b2i]).reshape(2, 1, d)
    eps_b = jnp.broadcast_to(jnp.reshape(eps, (1, 1)), (8, d))

    out_cat = pl.pallas_call(
        _mlp_body,
        grid=(2, half),
        in_specs=[
            pl.BlockSpec((1, blk, hd), lambda r, i: (2 * r, i, 0)),
            pl.BlockSpec((1, blk, hd), lambda r, i: (2 * r + 1, i, 0)),
            pl.BlockSpec((blk, d), lambda r, i: (r * half + i, 0)),
            pl.BlockSpec((8, d), lambda r, i: (0, 0)),
            pl.BlockSpec((1, d, d), lambda r, i: (r, 0, 0)),
            pl.BlockSpec((1, 1, d), lambda r, i: (r, 0, 0)),
            pl.BlockSpec((1, d, d), lambda r, i: (r, 0, 0)),
            pl.BlockSpec((1, 1, d), lambda r, i: (r, 0, 0)),
        ],
        out_specs=pl.BlockSpec((blk, d), lambda r, i: (r * half + i, 0)),
        out_shape=jax.ShapeDtypeStruct((2 * n, d), jnp.float32),
    )(hpre_h, hpre_h, x_cat, eps_b, w1s, b1s, w2s, b2s)

    return out_cat[:n], out_cat[n:]
